# baseline (speedup 1.0000x reference)
import functools

import jax
import jax.numpy as jnp
from jax import lax
from jax.experimental import pallas as pl
from jax.experimental.pallas import tpu as pltpu
from jax.experimental.pallas import tpu_sc as plsc

EMB = 64
NC = 2
NS = 16
NW = NC * NS
IDX_MINOR = 128
GATHERS_PER_CHUNK = 4
CHUNK = IDX_MINOR * GATHERS_PER_CHUNK


def _make_lookup(batch, hist):
    total = batch * hist
    assert total % (NW * IDX_MINOR) == 0
    rows_per_w = total // NW
    n_chunks = rows_per_w // CHUNK
    idx_rows_per_w = rows_per_w // IDX_MINOR

    mesh = plsc.VectorSubcoreMesh(
        core_axis_name="c", subcore_axis_name="s",
        num_cores=NC, num_subcores=NS,
    )

    @functools.partial(
        pl.kernel,
        out_type=jax.ShapeDtypeStruct((total, EMB), jnp.float32),
        mesh=mesh,
        scratch_types=[
            pltpu.VMEM((idx_rows_per_w, IDX_MINOR), jnp.int32),
            pltpu.VMEM((2, CHUNK, EMB), jnp.float32),
            pltpu.SemaphoreType.DMA((2,)),
            pltpu.SemaphoreType.DMA((2,)),
        ],
        compiler_params=pltpu.CompilerParams(use_tc_tiling_on_sc=False),
    )
    def lookup(tok_hbm, table_hbm, out_hbm, idx_v, rows_v, gsem, wsem):
        wid = lax.axis_index("s") * NC + lax.axis_index("c")
        base = wid * rows_per_w

        pltpu.sync_copy(tok_hbm.at[pl.ds(wid * idx_rows_per_w, idx_rows_per_w)],
                        idx_v)

        def gather_chunk(c, buf):
            row0 = c * GATHERS_PER_CHUNK
            for g in range(GATHERS_PER_CHUNK):
                pltpu.async_copy(
                    table_hbm.at[idx_v.at[row0 + g]],
                    rows_v.at[buf, pl.ds(g * IDX_MINOR, IDX_MINOR)],
                    gsem.at[buf],
                )

        def write_chunk(c, buf):
            pltpu.async_copy(
                rows_v.at[buf],
                out_hbm.at[pl.ds(base + c * CHUNK, CHUNK)],
                wsem.at[buf],
            )

        def drain_gathers(buf):
            for g in range(GATHERS_PER_CHUNK):
                pltpu.make_async_copy(
                    table_hbm.at[pl.ds(0, IDX_MINOR)],
                    rows_v.at[buf, pl.ds(0, IDX_MINOR)],
                    gsem.at[buf],
                ).wait()

        gather_chunk(0, 0)

        def step(c, _):
            buf = lax.rem(c, 2)
            nxt = 1 - buf
            @pl.when(c + 1 < n_chunks)
            def _():
                @pl.when(c >= 1)
                def _():
                    pltpu.make_async_copy(
                        rows_v.at[nxt],
                        out_hbm.at[pl.ds(base, CHUNK)],
                        wsem.at[nxt],
                    ).wait()
                gather_chunk(c + 1, nxt)
            drain_gathers(buf)
            write_chunk(c, buf)
            return ()

        lax.fori_loop(0, n_chunks, step, (), unroll=False)

        for buf in range(2):
            pltpu.make_async_copy(
                rows_v.at[buf],
                out_hbm.at[pl.ds(base, CHUNK)],
                wsem.at[buf],
            ).wait()

    return lookup


def kernel(token_ids, embeddings):
    batch, hist = token_ids.shape
    total = batch * hist
    tok = token_ids.reshape(total // IDX_MINOR, IDX_MINOR).astype(jnp.int32)
    out = _make_lookup(batch, hist)(tok, embeddings)
    return out.reshape(batch, hist, EMB)
